# Initial kernel scaffold; baseline (speedup 1.0000x reference)
import jax
import jax.numpy as jnp
from jax.experimental import pallas as pl


def kernel(node_embeds, edge_index_r0, edge_index_r1, edge_index_r2, rel_emb_r0, rel_emb_r1, rel_emb_r2):
    raise NotImplementedError("write your pallas kernel here")


# speedup vs baseline: 1.1723x; 1.1723x over previous
import functools

import jax
import jax.numpy as jnp
from jax import lax
from jax.experimental import pallas as pl
from jax.experimental.pallas import tpu as pltpu
from jax.experimental.pallas import tpu_sc as plsc

N_NODES = 10000
D = 128
E = 320000
NC = 2
NS = 16
NW = NC * NS
EPW = E // NW
CHUNK = 80
NCHUNK = EPW // CHUNK
LANES = 16
NG = D // LANES
EGRP = CHUNK // LANES


def _dismult_body(table, src0, dst0, src1, dst1, src2, dst2,
                  rel0, rel1, rel2,
                  out0, out1, out2,
                  idx_s_v, idx_d_v, rows_s_v, rows_d_v, scores_v, rel_v,
                  rel_sm, sem_s, sem_d):
    wid = lax.axis_index("s") * NC + lax.axis_index("c")
    base = wid * EPW
    iota = lax.iota(jnp.int32, LANES)

    for src, dst, rel, out in ((src0, dst0, rel0, out0),
                               (src1, dst1, rel1, out1),
                               (src2, dst2, rel2, out2)):
        pltpu.sync_copy(rel.at[:], rel_v)
        pltpu.sync_copy(src.at[pl.ds(base, EPW)], idx_s_v)
        pltpu.sync_copy(dst.at[pl.ds(base, EPW)], idx_d_v)
        for g in range(NG):
            vec = rel_v[pl.ds(g * LANES, LANES)]
            for i in range(LANES):
                rel_sm[g * LANES + i] = vec[i]

        def chunk_body(j, _):
            cs = j * CHUNK
            cp_s = pltpu.async_copy(
                table.at[idx_s_v.at[pl.ds(cs, CHUNK)]], rows_s_v, sem_s)
            cp_d = pltpu.async_copy(
                table.at[idx_d_v.at[pl.ds(cs, CHUNK)]], rows_d_v, sem_d)
            cp_s.wait()
            cp_d.wait()

            for k in range(EGRP):
                row = iota + (k * LANES)

                def col_body(dd, accs, row=row):
                    new = []
                    for g in range(NG):
                        d = g * LANES + dd
                        col = jnp.full((LANES,), d, dtype=jnp.int32)
                        a = plsc.load_gather(rows_s_v, [row, col])
                        b = plsc.load_gather(rows_d_v, [row, col])
                        new.append(accs[g] + a * b * rel_sm[d])
                    return tuple(new)

                accs = lax.fori_loop(
                    0, LANES, col_body,
                    tuple(jnp.zeros((LANES,), jnp.float32)
                          for _ in range(NG)))
                score = accs[0]
                for g in range(1, NG):
                    score = score + accs[g]
                scores_v[pl.ds(cs + k * LANES, LANES)] = score
            return 0

        lax.fori_loop(0, NCHUNK, chunk_body, 0)
        pltpu.sync_copy(scores_v, out.at[pl.ds(base, EPW)])


def kernel(node_embeds, edge_index_r0, edge_index_r1, edge_index_r2,
           rel_emb_r0, rel_emb_r1, rel_emb_r2):
    mesh = plsc.VectorSubcoreMesh(core_axis_name="c", subcore_axis_name="s")
    score_ty = jax.ShapeDtypeStruct((E,), jnp.float32)
    run = pl.kernel(
        _dismult_body,
        out_type=(score_ty, score_ty, score_ty),
        mesh=mesh,
        compiler_params=pltpu.CompilerParams(needs_layout_passes=False),
        scratch_types=[
            pltpu.VMEM((EPW,), jnp.int32),
            pltpu.VMEM((EPW,), jnp.int32),
            pltpu.VMEM((CHUNK, D), jnp.float32),
            pltpu.VMEM((CHUNK, D), jnp.float32),
            pltpu.VMEM((EPW,), jnp.float32),
            pltpu.VMEM((D,), jnp.float32),
            pltpu.SMEM((D,), jnp.float32),
            pltpu.SemaphoreType.DMA,
            pltpu.SemaphoreType.DMA,
        ],
    )
    return run(node_embeds,
               edge_index_r0[0], edge_index_r0[1],
               edge_index_r1[0], edge_index_r1[1],
               edge_index_r2[0], edge_index_r2[1],
               rel_emb_r0, rel_emb_r1, rel_emb_r2)


# speedup vs baseline: 1.3277x; 1.1326x over previous
import functools

import jax
import jax.numpy as jnp
from jax import lax
from jax.experimental import pallas as pl
from jax.experimental.pallas import tpu as pltpu
from jax.experimental.pallas import tpu_sc as plsc

N_NODES = 10000
D = 128
E = 320000
NC = 2
NS = 16
NW = NC * NS
EPW = E // NW
CHUNK = 80
NCHUNK = EPW // CHUNK
LANES = 16
NG = D // LANES
EGRP = CHUNK // LANES


def _dismult_body(table, src0, dst0, src1, dst1, src2, dst2,
                  rel0, rel1, rel2,
                  out0, out1, out2,
                  idx_s_v, idx_d_v, rows_s0, rows_d0, rows_s1, rows_d1,
                  scores_v, rel_v, rel_sm,
                  sem_s0, sem_d0, sem_s1, sem_d1):
    wid = lax.axis_index("s") * NC + lax.axis_index("c")
    base = wid * EPW
    iota = lax.iota(jnp.int32, LANES)
    bufs = ((rows_s0, rows_d0, sem_s0, sem_d0),
            (rows_s1, rows_d1, sem_s1, sem_d1))

    for src, dst, rel, out in ((src0, dst0, rel0, out0),
                               (src1, dst1, rel1, out1),
                               (src2, dst2, rel2, out2)):
        pltpu.sync_copy(rel.at[:], rel_v)
        pltpu.sync_copy(src.at[pl.ds(base, EPW)], idx_s_v)
        pltpu.sync_copy(dst.at[pl.ds(base, EPW)], idx_d_v)
        for g in range(NG):
            vec = rel_v[pl.ds(g * LANES, LANES)]
            for i in range(LANES):
                rel_sm[g * LANES + i] = vec[i]

        def fire(c, buf):
            rs, rd, ss, sd = buf
            cs = c * CHUNK
            pltpu.async_copy(table.at[idx_s_v.at[pl.ds(cs, CHUNK)]], rs, ss)
            pltpu.async_copy(table.at[idx_d_v.at[pl.ds(cs, CHUNK)]], rd, sd)

        def drain(buf):
            rs, rd, ss, sd = buf
            pltpu.make_async_copy(table.at[idx_s_v.at[pl.ds(0, CHUNK)]],
                                  rs, ss).wait()
            pltpu.make_async_copy(table.at[idx_d_v.at[pl.ds(0, CHUNK)]],
                                  rd, sd).wait()

        def compute(c, buf):
            rs, rd, _, _ = buf
            cs = c * CHUNK
            for k in range(EGRP):
                row = iota + (k * LANES)

                def col_body(dd, accs, row=row):
                    new = []
                    for g in range(NG):
                        d = g * LANES + dd
                        col = jnp.full((LANES,), d, dtype=jnp.int32)
                        a = plsc.load_gather(rs, [row, col])
                        b = plsc.load_gather(rd, [row, col])
                        new.append(accs[g] + a * b * rel_sm[d])
                    return tuple(new)

                accs = lax.fori_loop(
                    0, LANES, col_body,
                    tuple(jnp.zeros((LANES,), jnp.float32)
                          for _ in range(NG)))
                score = accs[0]
                for g in range(1, NG):
                    score = score + accs[g]
                scores_v[pl.ds(cs + k * LANES, LANES)] = score

        fire(0, bufs[0])

        def pair_body(jj, _):
            c = jj * 2
            fire(c + 1, bufs[1])
            drain(bufs[0])
            compute(c, bufs[0])
            fire(c + 2, bufs[0])
            drain(bufs[1])
            compute(c + 1, bufs[1])
            return 0

        lax.fori_loop(0, (NCHUNK - 1) // 2, pair_body, 0)
        drain(bufs[0])
        compute(NCHUNK - 1, bufs[0])
        pltpu.sync_copy(scores_v, out.at[pl.ds(base, EPW)])


def kernel(node_embeds, edge_index_r0, edge_index_r1, edge_index_r2,
           rel_emb_r0, rel_emb_r1, rel_emb_r2):
    mesh = plsc.VectorSubcoreMesh(core_axis_name="c", subcore_axis_name="s")
    score_ty = jax.ShapeDtypeStruct((E,), jnp.float32)
    run = pl.kernel(
        _dismult_body,
        out_type=(score_ty, score_ty, score_ty),
        mesh=mesh,
        compiler_params=pltpu.CompilerParams(needs_layout_passes=False),
        scratch_types=[
            pltpu.VMEM((EPW,), jnp.int32),
            pltpu.VMEM((EPW,), jnp.int32),
            pltpu.VMEM((CHUNK, D), jnp.float32),
            pltpu.VMEM((CHUNK, D), jnp.float32),
            pltpu.VMEM((CHUNK, D), jnp.float32),
            pltpu.VMEM((CHUNK, D), jnp.float32),
            pltpu.VMEM((EPW,), jnp.float32),
            pltpu.VMEM((D,), jnp.float32),
            pltpu.SMEM((D,), jnp.float32),
            pltpu.SemaphoreType.DMA,
            pltpu.SemaphoreType.DMA,
            pltpu.SemaphoreType.DMA,
            pltpu.SemaphoreType.DMA,
        ],
    )
    return run(node_embeds,
               edge_index_r0[0], edge_index_r0[1],
               edge_index_r1[0], edge_index_r1[1],
               edge_index_r2[0], edge_index_r2[1],
               rel_emb_r0, rel_emb_r1, rel_emb_r2)


# speedup vs baseline: 7.5767x; 5.7066x over previous
import functools

import jax
import jax.numpy as jnp
from jax import lax
from jax.experimental import pallas as pl
from jax.experimental.pallas import tpu as pltpu
from jax.experimental.pallas import tpu_sc as plsc

N_NODES = 10000
D = 128
E = 320000
NC = 2
NS = 16
NW = NC * NS
EPW = E // NW
CHUNK = 80
NCHUNK = EPW // CHUNK
LANES = 16
NG = D // LANES
EGRP = CHUNK // LANES


def _dismult_body(table, src0, dst0, src1, dst1, src2, dst2,
                  rel0, rel1, rel2,
                  out0, out1, out2,
                  idx_s_v, idx_d_v, rows_s0, rows_d0, rows_s1, rows_d1,
                  scores_v, rel_v, tmp_v,
                  sem_s0, sem_d0, sem_s1, sem_d1):
    wid = lax.axis_index("s") * NC + lax.axis_index("c")
    base = wid * EPW
    iota = lax.iota(jnp.int32, LANES)
    col_stride = iota * CHUNK
    bufs = ((rows_s0, rows_d0, sem_s0, sem_d0),
            (rows_s1, rows_d1, sem_s1, sem_d1))

    for src, dst, rel, out in ((src0, dst0, rel0, out0),
                               (src1, dst1, rel1, out1),
                               (src2, dst2, rel2, out2)):
        pltpu.sync_copy(rel.at[:], rel_v)
        pltpu.sync_copy(src.at[pl.ds(base, EPW)], idx_s_v)
        pltpu.sync_copy(dst.at[pl.ds(base, EPW)], idx_d_v)
        rel_regs = [rel_v[pl.ds(g * LANES, LANES)] for g in range(NG)]

        def fire(c, buf):
            rs, rd, ss, sd = buf
            cs = c * CHUNK
            pltpu.async_copy(table.at[idx_s_v.at[pl.ds(cs, CHUNK)]], rs, ss)
            pltpu.async_copy(table.at[idx_d_v.at[pl.ds(cs, CHUNK)]], rd, sd)

        def drain(buf):
            rs, rd, ss, sd = buf
            pltpu.make_async_copy(table.at[idx_s_v.at[pl.ds(0, CHUNK)]],
                                  rs, ss).wait()
            pltpu.make_async_copy(table.at[idx_d_v.at[pl.ds(0, CHUNK)]],
                                  rd, sd).wait()

        def compute(c, buf, rel_regs=rel_regs):
            rs, rd, _, _ = buf
            cs = c * CHUNK

            def edge_body(e, _):
                prods = []
                for g in range(NG):
                    a = rs[e, pl.ds(g * LANES, LANES)]
                    b = rd[e, pl.ds(g * LANES, LANES)]
                    prods.append(a * b * rel_regs[g])
                while len(prods) > 1:
                    prods = [x + y for x, y in zip(prods[::2], prods[1::2])]
                plsc.store_scatter(tmp_v, [col_stride + e], prods[0])
                return 0

            lax.fori_loop(0, CHUNK, edge_body, 0, unroll=4)

            for k in range(EGRP):
                ssum = tmp_v[pl.ds(k * LANES, LANES)]
                for l in range(1, LANES):
                    ssum = ssum + tmp_v[pl.ds(l * CHUNK + k * LANES, LANES)]
                scores_v[pl.ds(cs + k * LANES, LANES)] = ssum

        fire(0, bufs[0])

        def pair_body(jj, _):
            c = jj * 2
            fire(c + 1, bufs[1])
            drain(bufs[0])
            compute(c, bufs[0])
            fire(c + 2, bufs[0])
            drain(bufs[1])
            compute(c + 1, bufs[1])
            return 0

        lax.fori_loop(0, (NCHUNK - 1) // 2, pair_body, 0)
        drain(bufs[0])
        compute(NCHUNK - 1, bufs[0])
        pltpu.sync_copy(scores_v, out.at[pl.ds(base, EPW)])


def kernel(node_embeds, edge_index_r0, edge_index_r1, edge_index_r2,
           rel_emb_r0, rel_emb_r1, rel_emb_r2):
    mesh = plsc.VectorSubcoreMesh(core_axis_name="c", subcore_axis_name="s")
    score_ty = jax.ShapeDtypeStruct((E,), jnp.float32)
    run = pl.kernel(
        _dismult_body,
        out_type=(score_ty, score_ty, score_ty),
        mesh=mesh,
        compiler_params=pltpu.CompilerParams(needs_layout_passes=False),
        scratch_types=[
            pltpu.VMEM((EPW,), jnp.int32),
            pltpu.VMEM((EPW,), jnp.int32),
            pltpu.VMEM((CHUNK, D), jnp.float32),
            pltpu.VMEM((CHUNK, D), jnp.float32),
            pltpu.VMEM((CHUNK, D), jnp.float32),
            pltpu.VMEM((CHUNK, D), jnp.float32),
            pltpu.VMEM((EPW,), jnp.float32),
            pltpu.VMEM((D,), jnp.float32),
            pltpu.VMEM((LANES * CHUNK,), jnp.float32),
            pltpu.SemaphoreType.DMA,
            pltpu.SemaphoreType.DMA,
            pltpu.SemaphoreType.DMA,
            pltpu.SemaphoreType.DMA,
        ],
    )
    return run(node_embeds,
               edge_index_r0[0], edge_index_r0[1],
               edge_index_r1[0], edge_index_r1[1],
               edge_index_r2[0], edge_index_r2[1],
               rel_emb_r0, rel_emb_r1, rel_emb_r2)


# speedup vs baseline: 7.5838x; 1.0009x over previous
import functools

import jax
import jax.numpy as jnp
from jax import lax
from jax.experimental import pallas as pl
from jax.experimental.pallas import tpu as pltpu
from jax.experimental.pallas import tpu_sc as plsc

N_NODES = 10000
D = 128
E = 320000
NC = 2
NS = 16
NW = NC * NS
EPW = E // NW
CHUNK = 80
NCHUNK = EPW // CHUNK
LANES = 16
NG = D // LANES
EGRP = CHUNK // LANES


def _dismult_body(table, src0, dst0, src1, dst1, src2, dst2,
                  rel0, rel1, rel2,
                  out0, out1, out2,
                  idx_s_v, idx_d_v, rows_s0, rows_d0, rows_s1, rows_d1,
                  scores_v, rel_v, tmp_v,
                  sem_s0, sem_d0, sem_s1, sem_d1):
    wid = lax.axis_index("s") * NC + lax.axis_index("c")
    base = wid * EPW
    iota = lax.iota(jnp.int32, LANES)
    col_stride = iota * CHUNK
    bufs = ((rows_s0, rows_d0, sem_s0, sem_d0),
            (rows_s1, rows_d1, sem_s1, sem_d1))

    for src, dst, rel, out in ((src0, dst0, rel0, out0),
                               (src1, dst1, rel1, out1),
                               (src2, dst2, rel2, out2)):
        pltpu.sync_copy(rel.at[:], rel_v)
        pltpu.sync_copy(src.at[pl.ds(base, EPW)], idx_s_v)
        pltpu.sync_copy(dst.at[pl.ds(base, EPW)], idx_d_v)
        rel_regs = [rel_v[pl.ds(g * LANES, LANES)] for g in range(NG)]

        def fire(c, buf):
            rs, rd, ss, sd = buf
            cs = c * CHUNK
            pltpu.async_copy(table.at[idx_s_v.at[pl.ds(cs, CHUNK)]], rs, ss)
            pltpu.async_copy(table.at[idx_d_v.at[pl.ds(cs, CHUNK)]], rd, sd)

        def drain(buf):
            rs, rd, ss, sd = buf
            pltpu.make_async_copy(table.at[idx_s_v.at[pl.ds(0, CHUNK)]],
                                  rs, ss).wait()
            pltpu.make_async_copy(table.at[idx_d_v.at[pl.ds(0, CHUNK)]],
                                  rd, sd).wait()

        def compute(c, buf, rel_regs=rel_regs):
            rs, rd, _, _ = buf
            cs = c * CHUNK

            def edge_body(e, _):
                prods = []
                for g in range(NG):
                    a = rs[e, pl.ds(g * LANES, LANES)]
                    b = rd[e, pl.ds(g * LANES, LANES)]
                    prods.append(a * b * rel_regs[g])
                while len(prods) > 1:
                    prods = [x + y for x, y in zip(prods[::2], prods[1::2])]
                plsc.store_scatter(tmp_v, [col_stride + e], prods[0])
                return 0

            lax.fori_loop(0, CHUNK, edge_body, 0, unroll=8)

            for k in range(EGRP):
                ssum = tmp_v[pl.ds(k * LANES, LANES)]
                for l in range(1, LANES):
                    ssum = ssum + tmp_v[pl.ds(l * CHUNK + k * LANES, LANES)]
                scores_v[pl.ds(cs + k * LANES, LANES)] = ssum

        fire(0, bufs[0])

        def pair_body(jj, _):
            c = jj * 2
            fire(c + 1, bufs[1])
            drain(bufs[0])
            compute(c, bufs[0])
            fire(c + 2, bufs[0])
            drain(bufs[1])
            compute(c + 1, bufs[1])
            return 0

        lax.fori_loop(0, (NCHUNK - 1) // 2, pair_body, 0)
        drain(bufs[0])
        compute(NCHUNK - 1, bufs[0])
        pltpu.sync_copy(scores_v, out.at[pl.ds(base, EPW)])


def kernel(node_embeds, edge_index_r0, edge_index_r1, edge_index_r2,
           rel_emb_r0, rel_emb_r1, rel_emb_r2):
    mesh = plsc.VectorSubcoreMesh(core_axis_name="c", subcore_axis_name="s")
    score_ty = jax.ShapeDtypeStruct((E,), jnp.float32)
    run = pl.kernel(
        _dismult_body,
        out_type=(score_ty, score_ty, score_ty),
        mesh=mesh,
        compiler_params=pltpu.CompilerParams(needs_layout_passes=False),
        scratch_types=[
            pltpu.VMEM((EPW,), jnp.int32),
            pltpu.VMEM((EPW,), jnp.int32),
            pltpu.VMEM((CHUNK, D), jnp.float32),
            pltpu.VMEM((CHUNK, D), jnp.float32),
            pltpu.VMEM((CHUNK, D), jnp.float32),
            pltpu.VMEM((CHUNK, D), jnp.float32),
            pltpu.VMEM((EPW,), jnp.float32),
            pltpu.VMEM((D,), jnp.float32),
            pltpu.VMEM((LANES * CHUNK,), jnp.float32),
            pltpu.SemaphoreType.DMA,
            pltpu.SemaphoreType.DMA,
            pltpu.SemaphoreType.DMA,
            pltpu.SemaphoreType.DMA,
        ],
    )
    return run(node_embeds,
               edge_index_r0[0], edge_index_r0[1],
               edge_index_r1[0], edge_index_r1[1],
               edge_index_r2[0], edge_index_r2[1],
               rel_emb_r0, rel_emb_r1, rel_emb_r2)


# speedup vs baseline: 7.6211x; 1.0049x over previous
import functools

import jax
import jax.numpy as jnp
from jax import lax
from jax.experimental import pallas as pl
from jax.experimental.pallas import tpu as pltpu
from jax.experimental.pallas import tpu_sc as plsc

N_NODES = 10000
D = 128
E = 320000
NC = 2
NS = 16
NW = NC * NS
EPW = E // NW
CHUNK = 80
NCHUNK = EPW // CHUNK
LANES = 16
NG = D // LANES
EGRP = CHUNK // LANES
SEG = 25


def _dismult_body(table, src0, dst0, src1, dst1, src2, dst2,
                  rel0, rel1, rel2,
                  out0, out1, out2,
                  rows_s0, rows_d0, rows_s1, rows_d1,
                  ibuf_s0, ibuf_d0, ibuf_s1, ibuf_d1,
                  seg_v, rel_v, tmp_v, spm_table,
                  sem_s0, sem_d0, sem_s1, sem_d1, sem_i0, sem_i1):
    sid = lax.axis_index("s")
    wid = sid * NC + lax.axis_index("c")
    base = wid * EPW
    iota = lax.iota(jnp.int32, LANES)
    col_stride = iota * CHUNK
    rbufs = ((rows_s0, rows_d0, sem_s0, sem_d0),
             (rows_s1, rows_d1, sem_s1, sem_d1))
    ibufs = ((ibuf_s0, ibuf_d0, sem_i0), (ibuf_s1, ibuf_d1, sem_i1))

    rpw = 624
    pltpu.sync_copy(table.at[pl.ds(sid * rpw, rpw)],
                    spm_table.at[pl.ds(sid * rpw, rpw)])

    @pl.when(sid == 0)
    def _():
        pltpu.sync_copy(table.at[pl.ds(NS * rpw, N_NODES - NS * rpw)],
                        spm_table.at[pl.ds(NS * rpw, N_NODES - NS * rpw)])

    plsc.subcore_barrier()

    for src, dst, rel, out in ((src0, dst0, rel0, out0),
                               (src1, dst1, rel1, out1),
                               (src2, dst2, rel2, out2)):
        pltpu.sync_copy(rel.at[:], rel_v)
        rel_regs = [rel_v[pl.ds(g * LANES, LANES)] for g in range(NG)]

        def fire_idx(c, ib):
            bs, bd, sem = ib
            cc = jnp.minimum(c, NCHUNK - 1)
            cs = base + cc * CHUNK
            pltpu.async_copy(src.at[pl.ds(cs, CHUNK)], bs, sem)
            pltpu.async_copy(dst.at[pl.ds(cs, CHUNK)], bd, sem)

        def drain_idx(ib):
            bs, bd, sem = ib
            pltpu.make_async_copy(src.at[pl.ds(base, CHUNK)], bs, sem).wait()
            pltpu.make_async_copy(dst.at[pl.ds(base, CHUNK)], bd, sem).wait()

        def fire_rows(ib, rb):
            bs, bd, _ = ib
            rs, rd, ss, sd = rb
            pltpu.async_copy(spm_table.at[bs], rs, ss)
            pltpu.async_copy(spm_table.at[bd], rd, sd)

        def drain_rows(ib, rb):
            bs, bd, _ = ib
            rs, rd, ss, sd = rb
            pltpu.make_async_copy(spm_table.at[bs], rs, ss).wait()
            pltpu.make_async_copy(spm_table.at[bd], rd, sd).wait()

        def compute(c, rb, rel_regs=rel_regs):
            rs, rd, _, _ = rb
            lo = lax.rem(c, SEG) * CHUNK

            def edge_body(e, _):
                prods = []
                for g in range(NG):
                    a = rs[e, pl.ds(g * LANES, LANES)]
                    b = rd[e, pl.ds(g * LANES, LANES)]
                    prods.append(a * b * rel_regs[g])
                while len(prods) > 1:
                    prods = [x + y for x, y in zip(prods[::2], prods[1::2])]
                plsc.store_scatter(tmp_v, [col_stride + e], prods[0])
                return 0

            lax.fori_loop(0, CHUNK, edge_body, 0, unroll=4)

            for k in range(EGRP):
                ssum = tmp_v[pl.ds(k * LANES, LANES)]
                for l in range(1, LANES):
                    ssum = ssum + tmp_v[pl.ds(l * CHUNK + k * LANES, LANES)]
                seg_v[pl.ds(lo + k * LANES, LANES)] = ssum

            @pl.when(lax.rem(c, SEG) == SEG - 1)
            def _():
                pltpu.sync_copy(
                    seg_v, out.at[pl.ds(base + (c - (SEG - 1)) * CHUNK,
                                        SEG * CHUNK)])

        fire_idx(0, ibufs[0])
        drain_idx(ibufs[0])
        fire_rows(ibufs[0], rbufs[0])
        fire_idx(1, ibufs[1])

        def pair_body(jj, _):
            c = jj * 2
            drain_idx(ibufs[1])
            fire_rows(ibufs[1], rbufs[1])
            drain_rows(ibufs[0], rbufs[0])
            fire_idx(c + 2, ibufs[0])
            compute(c, rbufs[0])
            drain_idx(ibufs[0])
            fire_rows(ibufs[0], rbufs[0])
            drain_rows(ibufs[1], rbufs[1])
            fire_idx(c + 3, ibufs[1])
            compute(c + 1, rbufs[1])
            return 0

        lax.fori_loop(0, (NCHUNK - 1) // 2, pair_body, 0)
        drain_rows(ibufs[0], rbufs[0])
        compute(NCHUNK - 1, rbufs[0])
        drain_idx(ibufs[1])


def kernel(node_embeds, edge_index_r0, edge_index_r1, edge_index_r2,
           rel_emb_r0, rel_emb_r1, rel_emb_r2):
    mesh = plsc.VectorSubcoreMesh(core_axis_name="c", subcore_axis_name="s")
    score_ty = jax.ShapeDtypeStruct((E,), jnp.float32)
    run = pl.kernel(
        _dismult_body,
        out_type=(score_ty, score_ty, score_ty),
        mesh=mesh,
        compiler_params=pltpu.CompilerParams(needs_layout_passes=False),
        scratch_types=[
            pltpu.VMEM((CHUNK, D), jnp.float32),
            pltpu.VMEM((CHUNK, D), jnp.float32),
            pltpu.VMEM((CHUNK, D), jnp.float32),
            pltpu.VMEM((CHUNK, D), jnp.float32),
            pltpu.VMEM((CHUNK,), jnp.int32),
            pltpu.VMEM((CHUNK,), jnp.int32),
            pltpu.VMEM((CHUNK,), jnp.int32),
            pltpu.VMEM((CHUNK,), jnp.int32),
            pltpu.VMEM((SEG * CHUNK,), jnp.float32),
            pltpu.VMEM((D,), jnp.float32),
            pltpu.VMEM((LANES * CHUNK,), jnp.float32),
            pltpu.VMEM_SHARED((N_NODES, D), jnp.float32),
            pltpu.SemaphoreType.DMA,
            pltpu.SemaphoreType.DMA,
            pltpu.SemaphoreType.DMA,
            pltpu.SemaphoreType.DMA,
            pltpu.SemaphoreType.DMA,
            pltpu.SemaphoreType.DMA,
        ],
    )
    return run(node_embeds,
               edge_index_r0[0], edge_index_r0[1],
               edge_index_r1[0], edge_index_r1[1],
               edge_index_r2[0], edge_index_r2[1],
               rel_emb_r0, rel_emb_r1, rel_emb_r2)


# speedup vs baseline: 7.9418x; 1.0421x over previous
import functools

import jax
import jax.numpy as jnp
from jax import lax
from jax.experimental import pallas as pl
from jax.experimental.pallas import tpu as pltpu
from jax.experimental.pallas import tpu_sc as plsc

N_NODES = 10000
D = 128
E = 320000
NC = 2
NS = 16
NW = NC * NS
EPW = E // NW
CHUNK = 80
NCHUNK = EPW // CHUNK
LANES = 16
NG = D // LANES
EGRP = CHUNK // LANES
SEG = 25


def _dismult_body(table, src0, dst0, src1, dst1, src2, dst2,
                  rel0, rel1, rel2,
                  out0, out1, out2,
                  rows_s0, rows_d0, rows_s1, rows_d1,
                  ibuf_s0, ibuf_d0, ibuf_s1, ibuf_d1,
                  seg_v, rel_v, tmp_v, spm_table,
                  sem_s0, sem_d0, sem_s1, sem_d1, sem_i0, sem_i1):
    sid = lax.axis_index("s")
    wid = sid * NC + lax.axis_index("c")
    base = wid * EPW
    iota = lax.iota(jnp.int32, LANES)
    col_stride = iota * CHUNK
    rbufs = ((rows_s0, rows_d0, sem_s0, sem_d0),
             (rows_s1, rows_d1, sem_s1, sem_d1))
    ibufs = ((ibuf_s0, ibuf_d0, sem_i0), (ibuf_s1, ibuf_d1, sem_i1))

    rpw = 624
    pltpu.sync_copy(table.at[pl.ds(sid * rpw, rpw)],
                    spm_table.at[pl.ds(sid * rpw, rpw)])

    @pl.when(sid == 0)
    def _():
        pltpu.sync_copy(table.at[pl.ds(NS * rpw, N_NODES - NS * rpw)],
                        spm_table.at[pl.ds(NS * rpw, N_NODES - NS * rpw)])

    plsc.subcore_barrier()

    for src, dst, rel, out in ((src0, dst0, rel0, out0),
                               (src1, dst1, rel1, out1),
                               (src2, dst2, rel2, out2)):
        pltpu.sync_copy(rel.at[:], rel_v)
        rel_regs = []
        for g in range(NG // 2):
            for h in range(2):
                rel_regs.append(
                    plsc.load_gather(rel_v, [g * 32 + 2 * iota + h]))

        def fire_idx(c, ib):
            bs, bd, sem = ib
            cc = jnp.minimum(c, NCHUNK - 1)
            cs = base + cc * CHUNK
            pltpu.async_copy(src.at[pl.ds(cs, CHUNK)], bs, sem)
            pltpu.async_copy(dst.at[pl.ds(cs, CHUNK)], bd, sem)

        def drain_idx(ib):
            bs, bd, sem = ib
            pltpu.make_async_copy(src.at[pl.ds(base, CHUNK)], bs, sem).wait()
            pltpu.make_async_copy(dst.at[pl.ds(base, CHUNK)], bd, sem).wait()

        def fire_rows(ib, rb):
            bs, bd, _ = ib
            rs, rd, ss, sd = rb
            pltpu.async_copy(spm_table.at[bs], rs, ss)
            pltpu.async_copy(spm_table.at[bd], rd, sd)

        def drain_rows(ib, rb):
            bs, bd, _ = ib
            rs, rd, ss, sd = rb
            pltpu.make_async_copy(spm_table.at[bs], rs, ss).wait()
            pltpu.make_async_copy(spm_table.at[bd], rd, sd).wait()

        def compute(c, rb, rel_regs=rel_regs):
            rs, rd, _, _ = rb
            lo = lax.rem(c, SEG) * CHUNK

            def edge_body(e, _):
                prods = []
                for g in range(NG // 2):
                    a32 = plsc.bitcast(rs[e, pl.ds(g * LANES, LANES)],
                                       jnp.bfloat16)
                    b32 = plsc.bitcast(rd[e, pl.ds(g * LANES, LANES)],
                                       jnp.bfloat16)
                    a_ev, a_od = plsc.unpack(
                        a32, format=plsc.PackFormat.INTERLEAVED)
                    b_ev, b_od = plsc.unpack(
                        b32, format=plsc.PackFormat.INTERLEAVED)
                    prods.append(a_ev * b_ev * rel_regs[2 * g])
                    prods.append(a_od * b_od * rel_regs[2 * g + 1])
                while len(prods) > 1:
                    prods = [x + y for x, y in zip(prods[::2], prods[1::2])]
                plsc.store_scatter(tmp_v, [col_stride + e], prods[0])
                return 0

            lax.fori_loop(0, CHUNK, edge_body, 0, unroll=4)

            for k in range(EGRP):
                ssum = tmp_v[pl.ds(k * LANES, LANES)]
                for l in range(1, LANES):
                    ssum = ssum + tmp_v[pl.ds(l * CHUNK + k * LANES, LANES)]
                seg_v[pl.ds(lo + k * LANES, LANES)] = ssum

            @pl.when(lax.rem(c, SEG) == SEG - 1)
            def _():
                pltpu.sync_copy(
                    seg_v, out.at[pl.ds(base + (c - (SEG - 1)) * CHUNK,
                                        SEG * CHUNK)])

        fire_idx(0, ibufs[0])
        drain_idx(ibufs[0])
        fire_rows(ibufs[0], rbufs[0])
        fire_idx(1, ibufs[1])

        def pair_body(jj, _):
            c = jj * 2
            drain_idx(ibufs[1])
            fire_rows(ibufs[1], rbufs[1])
            drain_rows(ibufs[0], rbufs[0])
            fire_idx(c + 2, ibufs[0])
            compute(c, rbufs[0])
            drain_idx(ibufs[0])
            fire_rows(ibufs[0], rbufs[0])
            drain_rows(ibufs[1], rbufs[1])
            fire_idx(c + 3, ibufs[1])
            compute(c + 1, rbufs[1])
            return 0

        lax.fori_loop(0, (NCHUNK - 1) // 2, pair_body, 0)
        drain_rows(ibufs[0], rbufs[0])
        compute(NCHUNK - 1, rbufs[0])
        drain_idx(ibufs[1])


def kernel(node_embeds, edge_index_r0, edge_index_r1, edge_index_r2,
           rel_emb_r0, rel_emb_r1, rel_emb_r2):
    mesh = plsc.VectorSubcoreMesh(core_axis_name="c", subcore_axis_name="s")
    score_ty = jax.ShapeDtypeStruct((E,), jnp.float32)
    run = pl.kernel(
        _dismult_body,
        out_type=(score_ty, score_ty, score_ty),
        mesh=mesh,
        compiler_params=pltpu.CompilerParams(needs_layout_passes=False),
        scratch_types=[
            pltpu.VMEM((CHUNK, D), jnp.int32),
            pltpu.VMEM((CHUNK, D), jnp.int32),
            pltpu.VMEM((CHUNK, D), jnp.int32),
            pltpu.VMEM((CHUNK, D), jnp.int32),
            pltpu.VMEM((CHUNK,), jnp.int32),
            pltpu.VMEM((CHUNK,), jnp.int32),
            pltpu.VMEM((CHUNK,), jnp.int32),
            pltpu.VMEM((CHUNK,), jnp.int32),
            pltpu.VMEM((SEG * CHUNK,), jnp.float32),
            pltpu.VMEM((D,), jnp.float32),
            pltpu.VMEM((LANES * CHUNK,), jnp.float32),
            pltpu.VMEM_SHARED((N_NODES, D), jnp.int32),
            pltpu.SemaphoreType.DMA,
            pltpu.SemaphoreType.DMA,
            pltpu.SemaphoreType.DMA,
            pltpu.SemaphoreType.DMA,
            pltpu.SemaphoreType.DMA,
            pltpu.SemaphoreType.DMA,
        ],
    )
    packed = jax.lax.bitcast_convert_type(
        node_embeds.astype(jnp.bfloat16).reshape(N_NODES, D // 2, 2),
        jnp.int32)
    table_i32 = jnp.pad(packed, ((0, 0), (0, D - D // 2)))
    return run(table_i32,
               edge_index_r0[0], edge_index_r0[1],
               edge_index_r1[0], edge_index_r1[1],
               edge_index_r2[0], edge_index_r2[1],
               rel_emb_r0, rel_emb_r1, rel_emb_r2)


# speedup vs baseline: 8.4363x; 1.0623x over previous
import functools

import jax
import jax.numpy as jnp
from jax import lax
from jax.experimental import pallas as pl
from jax.experimental.pallas import tpu as pltpu
from jax.experimental.pallas import tpu_sc as plsc

N_NODES = 10000
D = 128
E = 320000
NC = 2
NS = 16
NW = NC * NS
EPW = E // NW
CHUNK = 80
NCHUNK = EPW // CHUNK
LANES = 16
NG = D // LANES
EGRP = CHUNK // LANES
SEG = 25


def _dismult_body(table, src0, dst0, src1, dst1, src2, dst2,
                  rel0, rel1, rel2,
                  out0, out1, out2,
                  rows_s0, rows_d0, rows_s1, rows_d1,
                  ibuf_s0, ibuf_d0, ibuf_s1, ibuf_d1,
                  seg_v, rel_v, tmp_v, spm_table,
                  sem_s0, sem_d0, sem_s1, sem_d1, sem_i0, sem_i1):
    sid = lax.axis_index("s")
    wid = sid * NC + lax.axis_index("c")
    base = wid * EPW
    iota = lax.iota(jnp.int32, LANES)
    col_stride = iota * CHUNK
    rbufs = ((rows_s0, rows_d0, sem_s0, sem_d0),
             (rows_s1, rows_d1, sem_s1, sem_d1))
    ibufs = ((ibuf_s0, ibuf_d0, sem_i0), (ibuf_s1, ibuf_d1, sem_i1))

    rpw = 624
    pltpu.sync_copy(table.at[pl.ds(sid * rpw, rpw)],
                    spm_table.at[pl.ds(sid * rpw, rpw)])

    @pl.when(sid == 0)
    def _():
        pltpu.sync_copy(table.at[pl.ds(NS * rpw, N_NODES - NS * rpw)],
                        spm_table.at[pl.ds(NS * rpw, N_NODES - NS * rpw)])

    plsc.subcore_barrier()

    for src, dst, rel, out in ((src0, dst0, rel0, out0),
                               (src1, dst1, rel1, out1),
                               (src2, dst2, rel2, out2)):
        pltpu.sync_copy(rel.at[:], rel_v)
        rel_regs = []
        for g in range(NG // 2):
            for h in range(2):
                rel_regs.append(
                    plsc.load_gather(rel_v, [g * 32 + 2 * iota + h]))

        def fire_idx(c, ib):
            bs, bd, sem = ib
            cc = jnp.minimum(c, NCHUNK - 1)
            cs = base + cc * CHUNK
            pltpu.async_copy(src.at[pl.ds(cs, CHUNK)], bs, sem)
            pltpu.async_copy(dst.at[pl.ds(cs, CHUNK)], bd, sem)

        def drain_idx(ib):
            bs, bd, sem = ib
            pltpu.make_async_copy(src.at[pl.ds(base, CHUNK)], bs, sem).wait()
            pltpu.make_async_copy(dst.at[pl.ds(base, CHUNK)], bd, sem).wait()

        def fire_rows(ib, rb):
            bs, bd, _ = ib
            rs, rd, ss, sd = rb
            pltpu.async_copy(spm_table.at[bs], rs, ss)
            pltpu.async_copy(spm_table.at[bd], rd, sd)

        def drain_rows(ib, rb):
            bs, bd, _ = ib
            rs, rd, ss, sd = rb
            pltpu.make_async_copy(spm_table.at[bs], rs, ss).wait()
            pltpu.make_async_copy(spm_table.at[bd], rd, sd).wait()

        def compute(c, rb, rel_regs=rel_regs):
            rs, rd, _, _ = rb
            lo = lax.rem(c, SEG) * CHUNK

            def edge_body(e, _):
                prods = []
                for g in range(NG // 2):
                    a32 = plsc.bitcast(rs[e, pl.ds(g * LANES, LANES)],
                                       jnp.bfloat16)
                    b32 = plsc.bitcast(rd[e, pl.ds(g * LANES, LANES)],
                                       jnp.bfloat16)
                    p_ev, p_od = plsc.unpack(
                        a32 * b32, format=plsc.PackFormat.INTERLEAVED)
                    prods.append(p_ev * rel_regs[2 * g])
                    prods.append(p_od * rel_regs[2 * g + 1])
                while len(prods) > 1:
                    prods = [x + y for x, y in zip(prods[::2], prods[1::2])]
                plsc.store_scatter(tmp_v, [col_stride + e], prods[0])
                return 0

            lax.fori_loop(0, CHUNK, edge_body, 0, unroll=4)

            for k in range(EGRP):
                ssum = tmp_v[pl.ds(k * LANES, LANES)]
                for l in range(1, LANES):
                    ssum = ssum + tmp_v[pl.ds(l * CHUNK + k * LANES, LANES)]
                seg_v[pl.ds(lo + k * LANES, LANES)] = ssum

            @pl.when(lax.rem(c, SEG) == SEG - 1)
            def _():
                pltpu.sync_copy(
                    seg_v, out.at[pl.ds(base + (c - (SEG - 1)) * CHUNK,
                                        SEG * CHUNK)])

        fire_idx(0, ibufs[0])
        drain_idx(ibufs[0])
        fire_rows(ibufs[0], rbufs[0])
        fire_idx(1, ibufs[1])

        def pair_body(jj, _):
            c = jj * 2
            drain_idx(ibufs[1])
            fire_rows(ibufs[1], rbufs[1])
            drain_rows(ibufs[0], rbufs[0])
            fire_idx(c + 2, ibufs[0])
            compute(c, rbufs[0])
            drain_idx(ibufs[0])
            fire_rows(ibufs[0], rbufs[0])
            drain_rows(ibufs[1], rbufs[1])
            fire_idx(c + 3, ibufs[1])
            compute(c + 1, rbufs[1])
            return 0

        lax.fori_loop(0, (NCHUNK - 1) // 2, pair_body, 0)
        drain_rows(ibufs[0], rbufs[0])
        compute(NCHUNK - 1, rbufs[0])
        drain_idx(ibufs[1])


def kernel(node_embeds, edge_index_r0, edge_index_r1, edge_index_r2,
           rel_emb_r0, rel_emb_r1, rel_emb_r2):
    mesh = plsc.VectorSubcoreMesh(core_axis_name="c", subcore_axis_name="s")
    score_ty = jax.ShapeDtypeStruct((E,), jnp.float32)
    run = pl.kernel(
        _dismult_body,
        out_type=(score_ty, score_ty, score_ty),
        mesh=mesh,
        compiler_params=pltpu.CompilerParams(needs_layout_passes=False),
        scratch_types=[
            pltpu.VMEM((CHUNK, D), jnp.int32),
            pltpu.VMEM((CHUNK, D), jnp.int32),
            pltpu.VMEM((CHUNK, D), jnp.int32),
            pltpu.VMEM((CHUNK, D), jnp.int32),
            pltpu.VMEM((CHUNK,), jnp.int32),
            pltpu.VMEM((CHUNK,), jnp.int32),
            pltpu.VMEM((CHUNK,), jnp.int32),
            pltpu.VMEM((CHUNK,), jnp.int32),
            pltpu.VMEM((SEG * CHUNK,), jnp.float32),
            pltpu.VMEM((D,), jnp.float32),
            pltpu.VMEM((LANES * CHUNK,), jnp.float32),
            pltpu.VMEM_SHARED((N_NODES, D), jnp.int32),
            pltpu.SemaphoreType.DMA,
            pltpu.SemaphoreType.DMA,
            pltpu.SemaphoreType.DMA,
            pltpu.SemaphoreType.DMA,
            pltpu.SemaphoreType.DMA,
            pltpu.SemaphoreType.DMA,
        ],
    )
    packed = jax.lax.bitcast_convert_type(
        node_embeds.astype(jnp.bfloat16).reshape(N_NODES, D // 2, 2),
        jnp.int32)
    table_i32 = jnp.pad(packed, ((0, 0), (0, D - D // 2)))
    return run(table_i32,
               edge_index_r0[0], edge_index_r0[1],
               edge_index_r1[0], edge_index_r1[1],
               edge_index_r2[0], edge_index_r2[1],
               rel_emb_r0, rel_emb_r1, rel_emb_r2)
